# baseline (speedup 1.0000x reference)
import functools

import jax
import jax.numpy as jnp
from jax import lax
from jax.experimental import pallas as pl
from jax.experimental.pallas import tpu as pltpu
from jax.experimental.pallas import tpu_sc as plsc

D_M = 128
T_LEN = 200
PASS0 = 104
N_WORKERS = 32
LANES = 16
CHUNKS_PER_T = D_M // LANES
GRP = 4


def _sc_body(x_hbm, pe_hbm, out_hbm, pe_v, buf0, buf1,
             psem, isem0, isem1, osem0, osem1):
    nc = lax.axis_size("c")
    wid = lax.axis_index("s") * nc + lax.axis_index("c")
    b_per_w = out_hbm.shape[0] // N_WORKERS
    base = wid * b_per_w
    n_groups = b_per_w // GRP

    bufs = (buf0, buf1)
    isems = (isem0, isem1)
    osems = (osem0, osem1)

    for hoff, hlen in ((0, PASS0), (PASS0, T_LEN - PASS0)):
        pe_h = pltpu.async_copy(
            pe_hbm.at[pl.ds(hoff, hlen)], pe_v.at[pl.ds(0, hlen)], psem)
        in_h = [None, None]
        out_h = [None, None]

        def start_in(g, sl, hoff=hoff, hlen=hlen):
            return [
                pltpu.async_copy(
                    x_hbm.at[base + g * GRP + j, pl.ds(hoff, hlen)],
                    bufs[sl].at[j, pl.ds(0, hlen)], isems[sl])
                for j in range(GRP)
            ]

        in_h[0] = start_in(0, 0)
        pe_h.wait()
        for g in range(n_groups):
            cur = g % 2
            nxt = 1 - cur
            if g + 1 < n_groups:
                if out_h[nxt] is not None:
                    for h in out_h[nxt]:
                        h.wait()
                in_h[nxt] = start_in(g + 1, nxt)
            for h in in_h[cur]:
                h.wait()
            buf = bufs[cur]

            def add_body(t, _, buf=buf):
                for c in range(CHUNKS_PER_T):
                    s = pl.ds(c * LANES, LANES)
                    pv = pe_v[t, s]
                    for j in range(GRP):
                        buf[j, t, s] = buf[j, t, s] + pv
                return 0

            lax.fori_loop(0, hlen, add_body, 0)
            out_h[cur] = [
                pltpu.async_copy(
                    buf.at[j, pl.ds(0, hlen)],
                    out_hbm.at[base + g * GRP + j, pl.ds(hoff, hlen)],
                    osems[cur])
                for j in range(GRP)
            ]
        for hs in out_h:
            if hs is not None:
                for h in hs:
                    h.wait()


def _sc_add(x, pe_t):
    B = x.shape[0]
    mesh = plsc.VectorSubcoreMesh(core_axis_name="c", subcore_axis_name="s")
    f = functools.partial(
        pl.kernel,
        out_type=jax.ShapeDtypeStruct((B, T_LEN, D_M), jnp.float32),
        mesh=mesh,
        scratch_types=[
            pltpu.VMEM((PASS0, D_M), jnp.float32),
            pltpu.VMEM((GRP, PASS0, D_M), jnp.float32),
            pltpu.VMEM((GRP, PASS0, D_M), jnp.float32),
            pltpu.SemaphoreType.DMA,
            pltpu.SemaphoreType.DMA,
            pltpu.SemaphoreType.DMA,
            pltpu.SemaphoreType.DMA,
            pltpu.SemaphoreType.DMA,
        ],
    )(_sc_body)
    return f(x, pe_t)


def kernel(x, pe):
    B, T, D = x.shape
    return _sc_add(x, pe[:T])
